# Initial kernel scaffold; baseline (speedup 1.0000x reference)
import jax
import jax.numpy as jnp
from jax.experimental import pallas as pl


def kernel(e1, rel, ent_real, ent_img, rel_real, rel_img, conv_w, conv_b, fc_w, fc_b, score_w, score_b):
    raise NotImplementedError("write your pallas kernel here")


# speedup vs baseline: 102.5786x; 102.5786x over previous
import jax
import jax.numpy as jnp
from jax.experimental import pallas as pl
from jax.experimental.pallas import tpu as pltpu

_NENT = 128
_NREL = 16
_KDIM = _NENT + _NREL
_TB = 2048


def _table_kernel(er_ref, ei_ref, rr_ref, ri_ref, cw_ref, cb_ref, fw_ref,
                  fb_ref, g_ref):
    def mlp(x, br):
        cw = cw_ref[br].astype(jnp.float32)
        fw = fw_ref[br].astype(jnp.float32)
        feat = jnp.dot(x, cw, preferred_element_type=jnp.float32) + cb_ref[br]
        feat = jnp.maximum(feat, 0.0)
        h = jnp.dot(feat, fw, preferred_element_type=jnp.float32) + fb_ref[br]
        return jnp.maximum(h, 0.0)

    er = mlp(er_ref[...], 0)
    ei = mlp(ei_ref[...], 1)
    rr = mlp(rr_ref[...], 2)
    ri = mlp(ri_ref[...], 3)

    ee = jnp.concatenate([er, er, -ei, ei], axis=1)
    rv = jnp.concatenate([rr, ri, ri, rr], axis=1)
    top = jnp.concatenate([ee, jnp.zeros((_NENT, 128), jnp.float32)], axis=1)
    bot = jnp.concatenate([jnp.zeros((_NREL, 128), jnp.float32), rv], axis=1)
    g_ref[...] = jnp.concatenate([top, bot], axis=0)


def _score_kernel(e1_ref, rel_ref, g_ref, sw_ref, sb_ref, o_ref):
    tb = e1_ref.shape[0]
    i1 = jax.lax.broadcasted_iota(jnp.int32, (tb, _NENT), 1)
    i2 = jax.lax.broadcasted_iota(jnp.int32, (tb, _NREL), 1)
    oh1 = jnp.where(e1_ref[...] == i1, 1.0, 0.0)
    oh2 = jnp.where(rel_ref[...] == i2, 1.0, 0.0)
    oh = jnp.concatenate([oh1, oh2], axis=1)
    gat = jnp.dot(oh, g_ref[...], preferred_element_type=jnp.float32)
    p = gat[:, :128] * gat[:, 128:]
    pred = jnp.dot(p, sw_ref[...], preferred_element_type=jnp.float32)
    pred = pred + sb_ref[...]
    o_ref[...] = jax.nn.sigmoid(pred)


def kernel(e1, rel, ent_real, ent_img, rel_real, rel_img, conv_w, conv_b,
           fc_w, fc_b, score_w, score_b):
    B = int(e1.shape[0])
    tb = _TB
    while B % tb:
        tb //= 2

    g = pl.pallas_call(
        _table_kernel,
        out_shape=jax.ShapeDtypeStruct((_KDIM, 256), jnp.float32),
    )(ent_real, ent_img, rel_real, rel_img, conv_w, conv_b, fc_w, fc_b)

    swp = jnp.concatenate([score_w[:64], score_w[:64]], axis=0)
    swp = swp.astype(jnp.float32)
    e1_2d = e1.reshape(B, 1).astype(jnp.int32)
    rel_2d = rel.reshape(B, 1).astype(jnp.int32)

    return pl.pallas_call(
        _score_kernel,
        out_shape=jax.ShapeDtypeStruct((B, _NENT), jnp.float32),
        grid=(B // tb,),
        in_specs=[
            pl.BlockSpec((tb, 1), lambda b: (b, 0)),
            pl.BlockSpec((tb, 1), lambda b: (b, 0)),
            pl.BlockSpec((_KDIM, 256), lambda b: (0, 0)),
            pl.BlockSpec((128, 128), lambda b: (0, 0)),
            pl.BlockSpec((1, 128), lambda b: (0, 0)),
        ],
        out_specs=pl.BlockSpec((tb, _NENT), lambda b: (b, 0)),
        compiler_params=pltpu.CompilerParams(
            dimension_semantics=("parallel",)),
    )(e1_2d, rel_2d, g, swp, score_b)


# speedup vs baseline: 269.6351x; 2.6286x over previous
import jax
import jax.numpy as jnp
from jax.experimental import pallas as pl
from jax.experimental.pallas import tpu as pltpu

_NENT = 128
_NREL = 16
_KDIM = _NENT + _NREL
_TB = 2048


def _table_kernel(er_ref, ei_ref, rr_ref, ri_ref, cw_ref, cb_ref, fw_ref,
                  fb_ref, g_ref):
    def mlp(x, br):
        cw = cw_ref[br].astype(jnp.float32)
        fw = fw_ref[br].astype(jnp.float32)
        feat = jnp.dot(x, cw, preferred_element_type=jnp.float32) + cb_ref[br]
        feat = jnp.maximum(feat, 0.0)
        h = jnp.dot(feat, fw, preferred_element_type=jnp.float32) + fb_ref[br]
        return jnp.maximum(h, 0.0)

    er = mlp(er_ref[...], 0)
    ei = mlp(ei_ref[...], 1)
    rr = mlp(rr_ref[...], 2)
    ri = mlp(ri_ref[...], 3)

    ee = jnp.concatenate([er, er, -ei, ei], axis=1)
    rv = jnp.concatenate([rr, ri, ri, rr], axis=1)
    top = jnp.concatenate([ee, jnp.zeros((_NENT, 128), jnp.float32)], axis=1)
    bot = jnp.concatenate([jnp.zeros((_NREL, 128), jnp.float32), rv], axis=1)
    g_ref[...] = jnp.concatenate([top, bot], axis=0)


def _score_kernel(e1_ref, rel_ref, g_ref, sw_ref, sb_ref, o_ref):
    tb = e1_ref.shape[2]
    e1 = e1_ref[0]
    rel = rel_ref[0]
    i1 = jax.lax.broadcasted_iota(jnp.int32, (_NENT, tb), 0)
    i2 = jax.lax.broadcasted_iota(jnp.int32, (_NREL, tb), 0)
    oh1 = jnp.where(i1 == e1, 1.0, 0.0)
    oh2 = jnp.where(i2 == rel, 1.0, 0.0)
    oh = jnp.concatenate([oh1, oh2], axis=0)
    gat = jax.lax.dot_general(oh, g_ref[...], (((0,), (0,)), ((), ())),
                              preferred_element_type=jnp.float32)
    p = gat[:, :128] * gat[:, 128:]
    pred = jnp.dot(p, sw_ref[...], preferred_element_type=jnp.float32)
    pred = pred + sb_ref[...]
    o_ref[...] = jax.nn.sigmoid(pred)


def kernel(e1, rel, ent_real, ent_img, rel_real, rel_img, conv_w, conv_b,
           fc_w, fc_b, score_w, score_b):
    B = int(e1.shape[0])
    tb = _TB
    while B % tb:
        tb //= 2

    g = pl.pallas_call(
        _table_kernel,
        out_shape=jax.ShapeDtypeStruct((_KDIM, 256), jnp.float32),
    )(ent_real, ent_img, rel_real, rel_img, conv_w, conv_b, fc_w, fc_b)

    swp = jnp.concatenate([score_w[:64], score_w[:64]], axis=0)
    swp = swp.astype(jnp.float32)
    e1_3d = e1.reshape(B // tb, 1, tb).astype(jnp.int32)
    rel_3d = rel.reshape(B // tb, 1, tb).astype(jnp.int32)

    return pl.pallas_call(
        _score_kernel,
        out_shape=jax.ShapeDtypeStruct((B, _NENT), jnp.float32),
        grid=(B // tb,),
        in_specs=[
            pl.BlockSpec((1, 1, tb), lambda b: (b, 0, 0)),
            pl.BlockSpec((1, 1, tb), lambda b: (b, 0, 0)),
            pl.BlockSpec((_KDIM, 256), lambda b: (0, 0)),
            pl.BlockSpec((128, 128), lambda b: (0, 0)),
            pl.BlockSpec((1, 128), lambda b: (0, 0)),
        ],
        out_specs=pl.BlockSpec((tb, _NENT), lambda b: (b, 0)),
        compiler_params=pltpu.CompilerParams(
            dimension_semantics=("parallel",)),
    )(e1_3d, rel_3d, g, swp, score_b)


# speedup vs baseline: 367.7096x; 1.3637x over previous
import jax
import jax.numpy as jnp
from jax.experimental import pallas as pl
from jax.experimental.pallas import tpu as pltpu

_NENT = 128
_NREL = 16
_KDIM = _NENT + _NREL
_TB = 4096


def _table_kernel(er_ref, ei_ref, rr_ref, ri_ref, cw_ref, cb_ref, fw_ref,
                  fb_ref, g_ref):
    def mlp(x, br):
        cw = cw_ref[br].astype(jnp.float32)
        fw = fw_ref[br].astype(jnp.float32)
        feat = jnp.dot(x, cw, preferred_element_type=jnp.float32) + cb_ref[br]
        feat = jnp.maximum(feat, 0.0)
        h = jnp.dot(feat, fw, preferred_element_type=jnp.float32) + fb_ref[br]
        return jnp.maximum(h, 0.0)

    er = mlp(er_ref[...], 0)
    ei = mlp(ei_ref[...], 1)
    rr = mlp(rr_ref[...], 2)
    ri = mlp(ri_ref[...], 3)

    ee = jnp.concatenate([er, er, -ei, ei], axis=1)
    rv = jnp.concatenate([rr, ri, ri, rr], axis=1)
    top = jnp.concatenate([ee, jnp.zeros((_NENT, 128), jnp.float32)], axis=1)
    bot = jnp.concatenate([jnp.zeros((_NREL, 128), jnp.float32), rv], axis=1)
    g_ref[...] = jnp.concatenate([top, bot], axis=0)


def _score_kernel(e1_ref, rel_ref, g_ref, sw_ref, sb_ref, o_ref):
    tb = e1_ref.shape[2]
    e1 = e1_ref[0]
    rel = rel_ref[0]
    i1 = jax.lax.broadcasted_iota(jnp.int32, (_NENT, tb), 0)
    i2 = jax.lax.broadcasted_iota(jnp.int32, (_NREL, tb), 0)
    oh1 = jnp.where(i1 == e1, 1.0, 0.0)
    oh2 = jnp.where(i2 == rel, 1.0, 0.0)
    oh = jnp.concatenate([oh1, oh2], axis=0)
    gat = jax.lax.dot_general(oh, g_ref[...], (((0,), (0,)), ((), ())),
                              preferred_element_type=jnp.float32)
    p = gat[:, :128] * gat[:, 128:]
    pred = jnp.dot(p, sw_ref[...], preferred_element_type=jnp.float32)
    pred = pred + sb_ref[...]
    o_ref[...] = jax.nn.sigmoid(pred)


def kernel(e1, rel, ent_real, ent_img, rel_real, rel_img, conv_w, conv_b,
           fc_w, fc_b, score_w, score_b):
    B = int(e1.shape[0])
    tb = _TB
    while B % tb:
        tb //= 2

    g = pl.pallas_call(
        _table_kernel,
        out_shape=jax.ShapeDtypeStruct((_KDIM, 256), jnp.float32),
    )(ent_real, ent_img, rel_real, rel_img, conv_w, conv_b, fc_w, fc_b)

    swp = jnp.concatenate([score_w[:64], score_w[:64]], axis=0)
    swp = swp.astype(jnp.float32)
    e1_3d = e1.reshape(B // tb, 1, tb).astype(jnp.int32)
    rel_3d = rel.reshape(B // tb, 1, tb).astype(jnp.int32)

    return pl.pallas_call(
        _score_kernel,
        out_shape=jax.ShapeDtypeStruct((B, _NENT), jnp.float32),
        grid=(B // tb,),
        in_specs=[
            pl.BlockSpec((1, 1, tb), lambda b: (b, 0, 0)),
            pl.BlockSpec((1, 1, tb), lambda b: (b, 0, 0)),
            pl.BlockSpec((_KDIM, 256), lambda b: (0, 0)),
            pl.BlockSpec((128, 128), lambda b: (0, 0)),
            pl.BlockSpec((1, 128), lambda b: (0, 0)),
        ],
        out_specs=pl.BlockSpec((tb, _NENT), lambda b: (b, 0)),
        compiler_params=pltpu.CompilerParams(
            dimension_semantics=("parallel",)),
    )(e1_3d, rel_3d, g, swp, score_b)


# speedup vs baseline: 412.4449x; 1.1217x over previous
import jax
import jax.numpy as jnp
from jax.experimental import pallas as pl
from jax.experimental.pallas import tpu as pltpu

_NENT = 128
_NREL = 16
_KDIM = _NENT + _NREL
_TB = 8192


def _table_kernel(er_ref, ei_ref, rr_ref, ri_ref, cw_ref, cb_ref, fw_ref,
                  fb_ref, g_ref):
    def mlp(x, br):
        cw = cw_ref[br].astype(jnp.float32)
        fw = fw_ref[br].astype(jnp.float32)
        feat = jnp.dot(x, cw, preferred_element_type=jnp.float32) + cb_ref[br]
        feat = jnp.maximum(feat, 0.0)
        h = jnp.dot(feat, fw, preferred_element_type=jnp.float32) + fb_ref[br]
        return jnp.maximum(h, 0.0)

    er = mlp(er_ref[...], 0)
    ei = mlp(ei_ref[...], 1)
    rr = mlp(rr_ref[...], 2)
    ri = mlp(ri_ref[...], 3)

    ee = jnp.concatenate([er, er, -ei, ei], axis=1)
    rv = jnp.concatenate([rr, ri, ri, rr], axis=1)
    top = jnp.concatenate([ee, jnp.zeros((_NENT, 128), jnp.float32)], axis=1)
    bot = jnp.concatenate([jnp.zeros((_NREL, 128), jnp.float32), rv], axis=1)
    g_ref[...] = jnp.concatenate([top, bot], axis=0)


def _score_kernel(e1_ref, rel_ref, g_ref, sw_ref, sb_ref, o_ref):
    tb = e1_ref.shape[2]
    e1 = e1_ref[0]
    rel = rel_ref[0]
    i1 = jax.lax.broadcasted_iota(jnp.int32, (_NENT, tb), 0)
    i2 = jax.lax.broadcasted_iota(jnp.int32, (_NREL, tb), 0)
    oh1 = jnp.where(i1 == e1, 1.0, 0.0)
    oh2 = jnp.where(i2 == rel, 1.0, 0.0)
    oh = jnp.concatenate([oh1, oh2], axis=0)
    gat = jax.lax.dot_general(oh, g_ref[...], (((0,), (0,)), ((), ())),
                              preferred_element_type=jnp.float32)
    p = gat[:, :128] * gat[:, 128:]
    pred = jnp.dot(p, sw_ref[...], preferred_element_type=jnp.float32)
    pred = pred + sb_ref[...]
    o_ref[...] = jax.nn.sigmoid(pred)


def kernel(e1, rel, ent_real, ent_img, rel_real, rel_img, conv_w, conv_b,
           fc_w, fc_b, score_w, score_b):
    B = int(e1.shape[0])
    tb = _TB
    while B % tb:
        tb //= 2

    g = pl.pallas_call(
        _table_kernel,
        out_shape=jax.ShapeDtypeStruct((_KDIM, 256), jnp.float32),
    )(ent_real, ent_img, rel_real, rel_img, conv_w, conv_b, fc_w, fc_b)

    swp = jnp.concatenate([score_w[:64], score_w[:64]], axis=0)
    swp = swp.astype(jnp.float32)
    e1_3d = e1.reshape(B // tb, 1, tb).astype(jnp.int32)
    rel_3d = rel.reshape(B // tb, 1, tb).astype(jnp.int32)

    return pl.pallas_call(
        _score_kernel,
        out_shape=jax.ShapeDtypeStruct((B, _NENT), jnp.float32),
        grid=(B // tb,),
        in_specs=[
            pl.BlockSpec((1, 1, tb), lambda b: (b, 0, 0)),
            pl.BlockSpec((1, 1, tb), lambda b: (b, 0, 0)),
            pl.BlockSpec((_KDIM, 256), lambda b: (0, 0)),
            pl.BlockSpec((128, 128), lambda b: (0, 0)),
            pl.BlockSpec((1, 128), lambda b: (0, 0)),
        ],
        out_specs=pl.BlockSpec((tb, _NENT), lambda b: (b, 0)),
        compiler_params=pltpu.CompilerParams(
            dimension_semantics=("parallel",)),
    )(e1_3d, rel_3d, g, swp, score_b)


# speedup vs baseline: 434.5745x; 1.0537x over previous
import jax
import jax.numpy as jnp
from jax.experimental import pallas as pl
from jax.experimental.pallas import tpu as pltpu

_NENT = 128
_NREL = 16
_KDIM = _NENT + _NREL
_TB = 16384


def _table_kernel(er_ref, ei_ref, rr_ref, ri_ref, cw_ref, cb_ref, fw_ref,
                  fb_ref, g_ref):
    def mlp(x, br):
        cw = cw_ref[br].astype(jnp.float32)
        fw = fw_ref[br].astype(jnp.float32)
        feat = jnp.dot(x, cw, preferred_element_type=jnp.float32) + cb_ref[br]
        feat = jnp.maximum(feat, 0.0)
        h = jnp.dot(feat, fw, preferred_element_type=jnp.float32) + fb_ref[br]
        return jnp.maximum(h, 0.0)

    er = mlp(er_ref[...], 0)
    ei = mlp(ei_ref[...], 1)
    rr = mlp(rr_ref[...], 2)
    ri = mlp(ri_ref[...], 3)

    ee = jnp.concatenate([er, er, -ei, ei], axis=1)
    rv = jnp.concatenate([rr, ri, ri, rr], axis=1)
    top = jnp.concatenate([ee, jnp.zeros((_NENT, 128), jnp.float32)], axis=1)
    bot = jnp.concatenate([jnp.zeros((_NREL, 128), jnp.float32), rv], axis=1)
    g_ref[...] = jnp.concatenate([top, bot], axis=0)


def _score_kernel(e1_ref, rel_ref, g_ref, sw_ref, sb_ref, o_ref):
    tb = e1_ref.shape[2]
    e1 = e1_ref[0]
    rel = rel_ref[0]
    i1 = jax.lax.broadcasted_iota(jnp.int32, (_NENT, tb), 0)
    i2 = jax.lax.broadcasted_iota(jnp.int32, (_NREL, tb), 0)
    oh1 = jnp.where(i1 == e1, 1.0, 0.0)
    oh2 = jnp.where(i2 == rel, 1.0, 0.0)
    oh = jnp.concatenate([oh1, oh2], axis=0)
    gat = jax.lax.dot_general(oh, g_ref[...], (((0,), (0,)), ((), ())),
                              preferred_element_type=jnp.float32)
    p = gat[:, :128] * gat[:, 128:]
    pred = jnp.dot(p, sw_ref[...], preferred_element_type=jnp.float32)
    pred = pred + sb_ref[...]
    o_ref[...] = jax.nn.sigmoid(pred)


def kernel(e1, rel, ent_real, ent_img, rel_real, rel_img, conv_w, conv_b,
           fc_w, fc_b, score_w, score_b):
    B = int(e1.shape[0])
    tb = _TB
    while B % tb:
        tb //= 2

    g = pl.pallas_call(
        _table_kernel,
        out_shape=jax.ShapeDtypeStruct((_KDIM, 256), jnp.float32),
    )(ent_real, ent_img, rel_real, rel_img, conv_w, conv_b, fc_w, fc_b)

    swp = jnp.concatenate([score_w[:64], score_w[:64]], axis=0)
    swp = swp.astype(jnp.float32)
    e1_3d = e1.reshape(B // tb, 1, tb).astype(jnp.int32)
    rel_3d = rel.reshape(B // tb, 1, tb).astype(jnp.int32)

    return pl.pallas_call(
        _score_kernel,
        out_shape=jax.ShapeDtypeStruct((B, _NENT), jnp.float32),
        grid=(B // tb,),
        in_specs=[
            pl.BlockSpec((1, 1, tb), lambda b: (b, 0, 0)),
            pl.BlockSpec((1, 1, tb), lambda b: (b, 0, 0)),
            pl.BlockSpec((_KDIM, 256), lambda b: (0, 0)),
            pl.BlockSpec((128, 128), lambda b: (0, 0)),
            pl.BlockSpec((1, 128), lambda b: (0, 0)),
        ],
        out_specs=pl.BlockSpec((tb, _NENT), lambda b: (b, 0)),
        compiler_params=pltpu.CompilerParams(
            dimension_semantics=("parallel",)),
    )(e1_3d, rel_3d, g, swp, score_b)


# speedup vs baseline: 444.8429x; 1.0236x over previous
import jax
import jax.numpy as jnp
from jax.experimental import pallas as pl
from jax.experimental.pallas import tpu as pltpu

_NENT = 128
_NREL = 16
_KDIM = _NENT + _NREL
_TB = 32768


def _table_kernel(er_ref, ei_ref, rr_ref, ri_ref, cw_ref, cb_ref, fw_ref,
                  fb_ref, g_ref):
    def mlp(x, br):
        cw = cw_ref[br].astype(jnp.float32)
        fw = fw_ref[br].astype(jnp.float32)
        feat = jnp.dot(x, cw, preferred_element_type=jnp.float32) + cb_ref[br]
        feat = jnp.maximum(feat, 0.0)
        h = jnp.dot(feat, fw, preferred_element_type=jnp.float32) + fb_ref[br]
        return jnp.maximum(h, 0.0)

    er = mlp(er_ref[...], 0)
    ei = mlp(ei_ref[...], 1)
    rr = mlp(rr_ref[...], 2)
    ri = mlp(ri_ref[...], 3)

    ee = jnp.concatenate([er, er, -ei, ei], axis=1)
    rv = jnp.concatenate([rr, ri, ri, rr], axis=1)
    top = jnp.concatenate([ee, jnp.zeros((_NENT, 128), jnp.float32)], axis=1)
    bot = jnp.concatenate([jnp.zeros((_NREL, 128), jnp.float32), rv], axis=1)
    g_ref[...] = jnp.concatenate([top, bot], axis=0)


def _score_kernel(e1_ref, rel_ref, g_ref, sw_ref, sb_ref, o_ref):
    tb = e1_ref.shape[2]
    e1 = e1_ref[0]
    rel = rel_ref[0]
    i1 = jax.lax.broadcasted_iota(jnp.int32, (_NENT, tb), 0)
    i2 = jax.lax.broadcasted_iota(jnp.int32, (_NREL, tb), 0)
    oh1 = jnp.where(i1 == e1, 1.0, 0.0)
    oh2 = jnp.where(i2 == rel, 1.0, 0.0)
    oh = jnp.concatenate([oh1, oh2], axis=0)
    gat = jax.lax.dot_general(oh, g_ref[...], (((0,), (0,)), ((), ())),
                              preferred_element_type=jnp.float32)
    p = gat[:, :128] * gat[:, 128:]
    pred = jnp.dot(p, sw_ref[...], preferred_element_type=jnp.float32)
    pred = pred + sb_ref[...]
    o_ref[...] = jax.nn.sigmoid(pred)


def kernel(e1, rel, ent_real, ent_img, rel_real, rel_img, conv_w, conv_b,
           fc_w, fc_b, score_w, score_b):
    B = int(e1.shape[0])
    tb = _TB
    while B % tb:
        tb //= 2

    g = pl.pallas_call(
        _table_kernel,
        out_shape=jax.ShapeDtypeStruct((_KDIM, 256), jnp.float32),
    )(ent_real, ent_img, rel_real, rel_img, conv_w, conv_b, fc_w, fc_b)

    swp = jnp.concatenate([score_w[:64], score_w[:64]], axis=0)
    swp = swp.astype(jnp.float32)
    e1_3d = e1.reshape(B // tb, 1, tb).astype(jnp.int32)
    rel_3d = rel.reshape(B // tb, 1, tb).astype(jnp.int32)

    return pl.pallas_call(
        _score_kernel,
        out_shape=jax.ShapeDtypeStruct((B, _NENT), jnp.float32),
        grid=(B // tb,),
        in_specs=[
            pl.BlockSpec((1, 1, tb), lambda b: (b, 0, 0)),
            pl.BlockSpec((1, 1, tb), lambda b: (b, 0, 0)),
            pl.BlockSpec((_KDIM, 256), lambda b: (0, 0)),
            pl.BlockSpec((128, 128), lambda b: (0, 0)),
            pl.BlockSpec((1, 128), lambda b: (0, 0)),
        ],
        out_specs=pl.BlockSpec((tb, _NENT), lambda b: (b, 0)),
        compiler_params=pltpu.CompilerParams(
            dimension_semantics=("parallel",)),
    )(e1_3d, rel_3d, g, swp, score_b)
